# baseline (speedup 1.0000x reference)
import functools

import jax
import jax.numpy as jnp
from jax import lax
from jax.experimental import pallas as pl
from jax.experimental.pallas import tpu as pltpu
from jax.experimental.pallas import tpu_sc as plsc

VOCAB = 100000
EMB = 64
BATCH = 1024

_NC = 2
_NS = 16
_NW = _NC * _NS
_BPW = BATCH // _NW

@functools.cache
def _make_sc_gather():
    mesh = plsc.VectorSubcoreMesh(core_axis_name="c", subcore_axis_name="s")

    @functools.partial(
        pl.kernel,
        mesh=mesh,
        out_type=jax.ShapeDtypeStruct((BATCH, 2 * EMB), jnp.float32),
        scratch_types=[
            pltpu.VMEM((_BPW,), jnp.int32),
            pltpu.VMEM((_BPW,), jnp.int32),
            pltpu.VMEM((_BPW, 2 * EMB), jnp.float32),
            pltpu.SemaphoreType.DMA,
        ],
    )
    def _sc_gather(pairs_hbm, idx_hbm, out_hbm, idx_v, pair_v, rows_v, sem):
        wid = lax.axis_index("s") * _NC + lax.axis_index("c")
        base = wid * _BPW
        pltpu.sync_copy(idx_hbm.at[pl.ds(base, _BPW)], idx_v)
        for c in range(_BPW // 16):
            sl = pl.ds(c * 16, 16)
            pair_v[sl] = lax.shift_right_logical(idx_v[sl], 1)
        pltpu.async_copy(pairs_hbm.at[pair_v], rows_v, sem).wait()
        pltpu.sync_copy(rows_v, out_hbm.at[pl.ds(base, _BPW)])

    return _sc_gather


_VT = 2048


def _proj_body(pairs_ref, odd_ref, w_ref, b_ref, out_ref):
    pr = pairs_ref[...]
    odd = odd_ref[...] != 0
    emb = jnp.where(odd, pr[:, EMB:], pr[:, :EMB])
    acc = lax.dot_general(
        emb,
        w_ref[...],
        (((1,), (1,)), ((), ())),
        preferred_element_type=jnp.float32,
    )
    out_ref[...] = acc + b_ref[...]


def _projection(pair_rows, odd2d, fc_w, fc_b2d):
    nv = pl.cdiv(VOCAB, _VT)
    return pl.pallas_call(
        _proj_body,
        grid=(nv,),
        in_specs=[
            pl.BlockSpec((BATCH, 2 * EMB), lambda j: (0, 0)),
            pl.BlockSpec((BATCH, 1), lambda j: (0, 0)),
            pl.BlockSpec((_VT, EMB), lambda j: (j, 0)),
            pl.BlockSpec((1, _VT), lambda j: (0, j)),
        ],
        out_specs=pl.BlockSpec((BATCH, _VT), lambda j: (0, j)),
        out_shape=jax.ShapeDtypeStruct((BATCH, VOCAB), jnp.float32),
        compiler_params=pltpu.CompilerParams(
            dimension_semantics=("arbitrary",),
        ),
    )(pair_rows, odd2d, fc_w, fc_b2d)


def kernel(words, emb_table, fc_w, fc_b):
    words = words.astype(jnp.int32)
    pairs = emb_table.reshape(VOCAB // 2, 2 * EMB)
    pair_rows = _make_sc_gather()(pairs, words)
    odd2d = (words & 1).reshape(BATCH, 1)
    return _projection(pair_rows, odd2d, fc_w, fc_b.reshape(1, VOCAB))


# speedup vs baseline: 1.0003x; 1.0003x over previous
import functools

import jax
import jax.numpy as jnp
from jax import lax
from jax.experimental import pallas as pl
from jax.experimental.pallas import tpu as pltpu
from jax.experimental.pallas import tpu_sc as plsc

VOCAB = 100000
EMB = 64
BATCH = 1024

_NC = 2
_NS = 16
_NW = _NC * _NS
_BPW = BATCH // _NW

@functools.cache
def _make_sc_gather():
    mesh = plsc.VectorSubcoreMesh(core_axis_name="c", subcore_axis_name="s")

    @functools.partial(
        pl.kernel,
        mesh=mesh,
        out_type=jax.ShapeDtypeStruct((BATCH, 2 * EMB), jnp.float32),
        scratch_types=[
            pltpu.VMEM((_BPW,), jnp.int32),
            pltpu.VMEM((_BPW,), jnp.int32),
            pltpu.VMEM((_BPW, 2 * EMB), jnp.float32),
            pltpu.SemaphoreType.DMA,
        ],
        compiler_params=pltpu.CompilerParams(use_tc_tiling_on_sc=True),
    )
    def _sc_gather(pairs_hbm, idx_hbm, out_hbm, idx_v, pair_v, rows_v, sem):
        wid = lax.axis_index("s") * _NC + lax.axis_index("c")
        base = wid * _BPW
        pltpu.sync_copy(idx_hbm.at[pl.ds(base, _BPW)], idx_v)
        for c in range(_BPW // 16):
            sl = pl.ds(c * 16, 16)
            pair_v[sl] = lax.shift_right_logical(idx_v[sl], 1)
        pltpu.async_copy(pairs_hbm.at[pair_v], rows_v, sem).wait()
        pltpu.sync_copy(rows_v, out_hbm.at[pl.ds(base, _BPW)])

    return _sc_gather


_VT = 2048


def _proj_body(pairs_ref, odd_ref, w_ref, b_ref, out_ref):
    pr = pairs_ref[...]
    odd = odd_ref[...] != 0
    emb = jnp.where(odd, pr[:, EMB:], pr[:, :EMB])
    acc = lax.dot_general(
        emb,
        w_ref[...],
        (((1,), (1,)), ((), ())),
        preferred_element_type=jnp.float32,
    )
    out_ref[...] = acc + b_ref[...]


def _projection(pair_rows, odd2d, fc_w, fc_b2d):
    nv = pl.cdiv(VOCAB, _VT)
    return pl.pallas_call(
        _proj_body,
        grid=(nv,),
        in_specs=[
            pl.BlockSpec((BATCH, 2 * EMB), lambda j: (0, 0)),
            pl.BlockSpec((BATCH, 1), lambda j: (0, 0)),
            pl.BlockSpec((_VT, EMB), lambda j: (j, 0)),
            pl.BlockSpec((1, _VT), lambda j: (0, j)),
        ],
        out_specs=pl.BlockSpec((BATCH, _VT), lambda j: (0, j)),
        out_shape=jax.ShapeDtypeStruct((BATCH, VOCAB), jnp.float32),
        compiler_params=pltpu.CompilerParams(
            dimension_semantics=("arbitrary",),
        ),
    )(pair_rows, odd2d, fc_w, fc_b2d)


def kernel(words, emb_table, fc_w, fc_b):
    words = words.astype(jnp.int32)
    pairs = emb_table.reshape(VOCAB // 2, 2 * EMB)
    pair_rows = _make_sc_gather()(pairs, words)
    odd2d = (words & 1).reshape(BATCH, 1)
    return _projection(pair_rows, odd2d, fc_w, fc_b.reshape(1, VOCAB))
